# speedup vs baseline: 13.7386x; 13.7386x over previous
import functools

import jax
import jax.numpy as jnp
from jax import lax
from jax.experimental import pallas as pl
from jax.experimental.pallas import tpu as pltpu
from jax.experimental.pallas import tpu_sc as plsc

NC = 2
NS = 16
NW = NC * NS
LANES = 16
B = 128


def _fill_vec(ref, val, n):
    v = jnp.full((LANES,), val, dtype=ref.dtype)

    def body(i, c):
        ref[pl.ds(i * LANES, LANES)] = v
        return c

    lax.fori_loop(0, n // LANES, body, 0)


def _deg_kernel(n_pad, epw, zs, dst_hbm, degp_hbm, idx_v, ones_v, z_v, deg_sh,
                sem):
    c = lax.axis_index("c")
    s = lax.axis_index("s")
    wid = s * NC + c

    _fill_vec(ones_v, 1.0, B)
    _fill_vec(z_v, 0.0, zs)
    pltpu.sync_copy(z_v, deg_sh.at[pl.ds(s * zs, zs)])
    plsc.subcore_barrier()

    base = wid * epw

    def chunk(ci, carry):
        pltpu.sync_copy(dst_hbm.at[pl.ds(base + ci * B, B)], idx_v)
        pltpu.sync_copy(ones_v, deg_sh.at[idx_v], add=True)
        return carry

    lax.fori_loop(0, epw // B, chunk, 0)
    plsc.subcore_barrier()
    pltpu.sync_copy(deg_sh.at[pl.ds(s * zs, zs)],
                    degp_hbm.at[pl.ds(c * n_pad + s * zs, zs)])


def _edge_kernel(n_pad, epw, zs, d, src_hbm, dst_hbm, xs_hbm, outp_hbm,
                 sidx_v, didx_v, rows_v, z_v, out_sh, sem):
    c = lax.axis_index("c")
    s = lax.axis_index("s")
    wid = s * NC + c

    def zrow(i, carry):
        def zcol(j, cc):
            z_v[i, pl.ds(j * LANES, LANES)] = jnp.zeros((LANES,), jnp.float32)
            return cc

        lax.fori_loop(0, d // LANES, zcol, 0)
        return carry

    lax.fori_loop(0, 64, zrow, 0)

    def zcopy(t, carry):
        pltpu.sync_copy(z_v, out_sh.at[pl.ds(s * zs + t * 64, 64)])
        return carry

    lax.fori_loop(0, zs // 64, zcopy, 0)
    plsc.subcore_barrier()

    base = wid * epw

    def chunk(ci, carry):
        pltpu.sync_copy(src_hbm.at[pl.ds(base + ci * B, B)], sidx_v)
        pltpu.sync_copy(dst_hbm.at[pl.ds(base + ci * B, B)], didx_v)
        pltpu.async_copy(xs_hbm.at[sidx_v], rows_v, sem).wait()
        pltpu.sync_copy(rows_v, out_sh.at[didx_v], add=True)
        return carry

    lax.fori_loop(0, epw // B, chunk, 0)
    plsc.subcore_barrier()
    pltpu.sync_copy(out_sh.at[pl.ds(s * zs, zs)],
                    outp_hbm.at[pl.ds(c * n_pad + s * zs, zs)])


def _dis_from_parts(dp_ref):
    deg = dp_ref[0, :] + dp_ref[1, :]
    return jnp.where(deg > 0, lax.rsqrt(jnp.maximum(deg, 1.0)), 0.0)


def _scale_kernel(dp_ref, x_ref, xs_ref):
    dis = _dis_from_parts(dp_ref)
    xs_ref[...] = x_ref[...] * dis[:, None]


def _combine_kernel(p_ref, dp_ref, o_ref):
    dis = _dis_from_parts(dp_ref)
    o_ref[...] = (p_ref[0] + p_ref[1]) * dis[:, None]


@jax.jit
def kernel(x, edge_index):
    n, d = x.shape
    e = edge_index.shape[1]

    n_pad = ((n + NS * LANES - 1) // (NS * LANES)) * (NS * LANES)
    zs = n_pad // NS
    epw = ((e + NW * B - 1) // (NW * B)) * B
    e_pad = epw * NW
    sac = n_pad - 1

    src = edge_index[0]
    dst = edge_index[1]
    src_p = jnp.full((e_pad,), sac, jnp.int32).at[:e].set(src)
    dst_p = jnp.full((e_pad,), sac, jnp.int32).at[:e].set(dst)
    x_pad = jnp.zeros((n_pad, d), x.dtype).at[:n].set(x)

    mesh = plsc.VectorSubcoreMesh(core_axis_name="c", subcore_axis_name="s",
                                  num_cores=NC, num_subcores=NS)

    deg_parts = pl.kernel(
        functools.partial(_deg_kernel, n_pad, epw, zs),
        out_type=jax.ShapeDtypeStruct((NC * n_pad,), jnp.float32),
        mesh=mesh,
        scratch_types=[
            pltpu.VMEM((B,), jnp.int32),
            pltpu.VMEM((B,), jnp.float32),
            pltpu.VMEM((zs,), jnp.float32),
            pltpu.VMEM_SHARED((n_pad,), jnp.float32),
            pltpu.SemaphoreType.DMA,
        ],
    )(dst_p)
    deg_parts = deg_parts.reshape(NC, n_pad)

    rb = 1024
    grid = n_pad // rb
    xs = pl.pallas_call(
        _scale_kernel,
        grid=(grid,),
        in_specs=[
            pl.BlockSpec((NC, rb), lambda i: (0, i)),
            pl.BlockSpec((rb, d), lambda i: (i, 0)),
        ],
        out_specs=pl.BlockSpec((rb, d), lambda i: (i, 0)),
        out_shape=jax.ShapeDtypeStruct((n_pad, d), jnp.float32),
    )(deg_parts, x_pad)

    out_parts = pl.kernel(
        functools.partial(_edge_kernel, n_pad, epw, zs, d),
        out_type=jax.ShapeDtypeStruct((NC * n_pad, d), jnp.float32),
        mesh=mesh,
        scratch_types=[
            pltpu.VMEM((B,), jnp.int32),
            pltpu.VMEM((B,), jnp.int32),
            pltpu.VMEM((B, d), jnp.float32),
            pltpu.VMEM((64, d), jnp.float32),
            pltpu.VMEM_SHARED((n_pad, d), jnp.float32),
            pltpu.SemaphoreType.DMA,
        ],
    )(src_p, dst_p, xs)
    out_parts = out_parts.reshape(NC, n_pad, d)

    out_pad = pl.pallas_call(
        _combine_kernel,
        grid=(grid,),
        in_specs=[
            pl.BlockSpec((NC, rb, d), lambda i: (0, i, 0)),
            pl.BlockSpec((NC, rb), lambda i: (0, i)),
        ],
        out_specs=pl.BlockSpec((rb, d), lambda i: (i, 0)),
        out_shape=jax.ShapeDtypeStruct((n_pad, d), jnp.float32),
    )(out_parts, deg_parts)

    return out_pad[:n]
